# baseline (speedup 1.0000x reference)
import functools

import jax
import jax.numpy as jnp
from jax import lax
from jax.experimental import pallas as pl
from jax.experimental.pallas import tpu as pltpu
from jax.experimental.pallas import tpu_sc as plsc

_B, _P, _C = 32, 8732, 81
_PBLK = 4608
_NP = 2
_P_PAD = _PBLK * _NP
_CHUNKS = _P_PAD // 16
_UNROLL = 8
_NEGPOS = 3


def _dense_body(conf_ref, lab_ref, d4_ref, w_ref, wb_ref, stats_ref):
    p = pl.program_id(1)
    conf = jnp.transpose(conf_ref[0], (1, 0))
    lab = lab_ref[0]
    cols = lax.broadcasted_iota(jnp.int32, (1, _PBLK), 1) + p * _PBLK
    valid = cols < _P
    m = jnp.max(conf, axis=0, keepdims=True)
    e = jnp.exp(conf - m)
    lse = jnp.log(jnp.sum(e, axis=0, keepdims=True)) + m
    cls = lax.broadcasted_iota(jnp.int32, (_C, _PBLK), 0)
    picked = jnp.sum(jnp.where(cls == lab, conf, 0.0), axis=0, keepdims=True)
    ce = lse - picked
    isp = lab != 0
    posm = isp & valid
    w = jnp.where(valid & jnp.logical_not(isp), ce, 0.0)
    w_ref[0] = w
    wb_ref[0] = lax.bitcast_convert_type(w, jnp.int32)

    np_p = jnp.sum(jnp.where(posm, 1.0, 0.0))
    pce_p = jnp.sum(jnp.where(posm, ce, 0.0))
    d = d4_ref[0]
    ad = jnp.abs(d)
    sl1 = jnp.where(ad < 1.0, 0.5 * d * d, ad - 0.5)
    ll_p = jnp.sum(sl1)
    li = lax.broadcasted_iota(jnp.int32, (1, 1, 128), 2)
    partial = jnp.where(
        li == 0, np_p,
        jnp.where(li == 1, pce_p, jnp.where(li == 2, ll_p, 0.0)))

    @pl.when(p == 0)
    def _init():
        stats_ref[...] = jnp.zeros((1, 1, 128), jnp.float32)

    stats_ref[...] += partial

    @pl.when(p == _NP - 1)
    def _finish():
        np_i = stats_ref[0, 0, 0].astype(jnp.int32)
        k = jnp.clip(_NEGPOS * np_i, 1, _P - 1)
        j = jnp.minimum(k, _P - np_i)
        stats_ref[...] = jnp.where(li == 3, j.astype(jnp.float32),
                                   stats_ref[...])


def _dense_pass(conf_data, lab3, d4):
    return pl.pallas_call(
        _dense_body,
        grid=(_B, _NP),
        in_specs=[
            pl.BlockSpec((1, _PBLK, _C), lambda b, p: (b, p, 0)),
            pl.BlockSpec((1, 1, _PBLK), lambda b, p: (b, 0, p)),
            pl.BlockSpec((1, 1, 4 * _PBLK), lambda b, p: (b, 0, p)),
        ],
        out_specs=[
            pl.BlockSpec((1, 1, _PBLK), lambda b, p: (b, 0, p)),
            pl.BlockSpec((1, 1, _PBLK), lambda b, p: (b, 0, p)),
            pl.BlockSpec((1, 1, 128), lambda b, p: (b, 0, 0)),
        ],
        out_shape=[
            jax.ShapeDtypeStruct((_B, 1, _P_PAD), jnp.float32),
            jax.ShapeDtypeStruct((_B, 1, _P_PAD), jnp.int32),
            jax.ShapeDtypeStruct((_B, 1, 128), jnp.float32),
        ],
    )(conf_data, lab3, d4)


def _topk_body(w_hbm, wb_hbm, j_hbm, out_hbm, meta_hbm, w_v, wi_v, j_v, o_v, m_v):
    wid = lax.axis_index("s") * 2 + lax.axis_index("c")
    pltpu.sync_copy(w_hbm.at[wid], w_v)
    pltpu.sync_copy(wb_hbm.at[wid], wi_v)
    pltpu.sync_copy(j_hbm.at[wid], j_v)
    jv = j_v[...]
    onev = jnp.full((16,), 1, jnp.int32)

    def bit_step(i, ansv):
        candv = ansv | jnp.left_shift(onev, 30 - i)

        def chunk(c, cntv):
            for u in range(_UNROLL):
                wb = wi_v[pl.ds((c * _UNROLL + u) * 16, 16)]
                cntv = cntv + plsc.all_reduce_population_count(wb >= candv)
            return cntv

        cntv = lax.fori_loop(0, _CHUNKS // _UNROLL, chunk,
                             jnp.zeros((16,), jnp.int32))
        return jnp.where(cntv >= jv, candv, ansv)

    ansv = lax.fori_loop(0, 31, bit_step, jnp.zeros((16,), jnp.int32))

    def chunk2(c, carry):
        sacc, caccv = carry
        for u in range(_UNROLL):
            off = (c * _UNROLL + u) * 16
            wb = wi_v[pl.ds(off, 16)]
            gt = wb > ansv
            sacc = sacc + jnp.where(gt, w_v[pl.ds(off, 16)], 0.0)
            caccv = caccv + plsc.all_reduce_population_count(gt)
        return sacc, caccv

    sacc, caccv = lax.fori_loop(
        0, _CHUNKS // _UNROLL, chunk2,
        (jnp.zeros((16,), jnp.float32), jnp.zeros((16,), jnp.int32)))
    o_v[...] = sacc
    m_v[pl.ds(0, 16)] = ansv
    m_v[pl.ds(16, 16)] = caccv
    pltpu.sync_copy(o_v, out_hbm.at[wid])
    pltpu.sync_copy(m_v, meta_hbm.at[wid])


def _topk_pass(w_pad, wb_pad, j2):
    fn = pl.kernel(
        _topk_body,
        out_type=(
            jax.ShapeDtypeStruct((_B, 16), jnp.float32),
            jax.ShapeDtypeStruct((_B, 32), jnp.int32),
        ),
        mesh=plsc.VectorSubcoreMesh(core_axis_name="c", subcore_axis_name="s"),
        compiler_params=pltpu.CompilerParams(needs_layout_passes=False),
        scratch_types=[
            pltpu.VMEM((_P_PAD,), jnp.float32),
            pltpu.VMEM((_P_PAD,), jnp.int32),
            pltpu.VMEM((16,), jnp.int32),
            pltpu.VMEM((16,), jnp.float32),
            pltpu.VMEM((32,), jnp.int32),
        ],
    )
    return fn(w_pad, wb_pad, j2)


@jax.jit
def kernel(loc_data, conf_data, loc_t, conf_t):
    lab = conf_t.astype(jnp.int32)
    lab3 = jnp.pad(lab, ((0, 0), (0, _P_PAD - _P))).reshape(_B, 1, _P_PAD)
    pad4 = 4 * _P_PAD - 4 * _P
    d4 = jnp.where((lab != 0)[:, :, None], loc_data - loc_t, 0.0)
    d4 = jnp.pad(d4.reshape(_B, 4 * _P),
                 ((0, 0), (0, pad4))).reshape(_B, 1, 4 * _P_PAD)
    w3, wb3, stats = _dense_pass(conf_data, lab3, d4)
    stats = stats[:, 0, :]
    w_pad = w3.reshape(_B, _P_PAD)
    wb_pad = wb3.reshape(_B, _P_PAD)
    j = jnp.round(stats[:, 3]).astype(jnp.int32)
    j2 = jnp.broadcast_to(j[:, None], (_B, 16)) + jnp.zeros((_B, 16), jnp.int32)
    srows, meta = _topk_pass(w_pad, wb_pad, j2)
    ans = meta[:, 0]
    cnt = meta[:, 16]
    tau = lax.bitcast_convert_type(ans, jnp.float32)
    s_row = jnp.sum(srows, axis=1) + jnp.where(
        j > cnt, (j - cnt).astype(jnp.float32) * tau, 0.0)
    num_pos = stats[:, 0]
    n = jnp.maximum(jnp.sum(num_pos), 1.0)
    loss_l = jnp.sum(stats[:, 2]) / n
    loss_c = (jnp.sum(stats[:, 1]) + jnp.sum(s_row)) / n
    return (loss_l, loss_c)


# speedup vs baseline: 1.0761x; 1.0761x over previous
import functools

import jax
import jax.numpy as jnp
from jax import lax
from jax.experimental import pallas as pl
from jax.experimental.pallas import tpu as pltpu
from jax.experimental.pallas import tpu_sc as plsc

_B, _P, _C = 32, 8732, 81
_PBLK = 9216
_NP = 1
_P_PAD = _PBLK * _NP
_CHUNKS = _P_PAD // 16
_UNROLL = 8
_NEGPOS = 3


def _dense_body(conf_ref, lab_ref, d4_ref, w_ref, wb_ref, stats_ref):
    p = pl.program_id(1)
    conf = jnp.transpose(conf_ref[0], (1, 0))
    lab = lab_ref[0]
    cols = lax.broadcasted_iota(jnp.int32, (1, _PBLK), 1) + p * _PBLK
    valid = cols < _P
    m = jnp.max(conf, axis=0, keepdims=True)
    e = jnp.exp(conf - m)
    lse = jnp.log(jnp.sum(e, axis=0, keepdims=True)) + m
    cls = lax.broadcasted_iota(jnp.int32, (_C, _PBLK), 0)
    picked = jnp.sum(jnp.where(cls == lab, conf, 0.0), axis=0, keepdims=True)
    ce = lse - picked
    isp = lab != 0
    posm = isp & valid
    w = jnp.where(valid & jnp.logical_not(isp), ce, 0.0)
    w_ref[0] = w
    wb_ref[0] = lax.bitcast_convert_type(w, jnp.int32)

    np_p = jnp.sum(jnp.where(posm, 1.0, 0.0))
    pce_p = jnp.sum(jnp.where(posm, ce, 0.0))
    cols4 = lax.broadcasted_iota(jnp.int32, (1, 4 * _PBLK), 1) + p * 4 * _PBLK
    d = jnp.where(cols4 < 4 * _P, d4_ref[0], 0.0)
    ad = jnp.abs(d)
    sl1 = jnp.where(ad < 1.0, 0.5 * d * d, ad - 0.5)
    ll_p = jnp.sum(sl1)
    li = lax.broadcasted_iota(jnp.int32, (1, 1, 128), 2)
    partial = jnp.where(
        li == 0, np_p,
        jnp.where(li == 1, pce_p, jnp.where(li == 2, ll_p, 0.0)))

    @pl.when(p == 0)
    def _init():
        stats_ref[...] = jnp.zeros((1, 1, 128), jnp.float32)

    stats_ref[...] += partial

    @pl.when(p == _NP - 1)
    def _finish():
        np_i = stats_ref[0, 0, 0].astype(jnp.int32)
        k = jnp.clip(_NEGPOS * np_i, 1, _P - 1)
        j = jnp.minimum(k, _P - np_i)
        stats_ref[...] = jnp.where(li == 3, j.astype(jnp.float32),
                                   stats_ref[...])


def _dense_pass(conf_data, lab3, d4):
    return pl.pallas_call(
        _dense_body,
        grid=(_B, _NP),
        in_specs=[
            pl.BlockSpec((1, _PBLK, _C), lambda b, p: (b, p, 0)),
            pl.BlockSpec((1, 1, _PBLK), lambda b, p: (b, 0, p)),
            pl.BlockSpec((1, 1, 4 * _PBLK), lambda b, p: (b, 0, p)),
        ],
        out_specs=[
            pl.BlockSpec((1, 1, _PBLK), lambda b, p: (b, 0, p)),
            pl.BlockSpec((1, 1, _PBLK), lambda b, p: (b, 0, p)),
            pl.BlockSpec((1, 1, 128), lambda b, p: (b, 0, 0)),
        ],
        out_shape=[
            jax.ShapeDtypeStruct((_B, 1, _P_PAD), jnp.float32),
            jax.ShapeDtypeStruct((_B, 1, _P_PAD), jnp.int32),
            jax.ShapeDtypeStruct((_B, 1, 128), jnp.float32),
        ],
    )(conf_data, lab3, d4)


def _topk_body(w_hbm, wb_hbm, j_hbm, out_hbm, meta_hbm, w_v, wi_v, j_v, o_v, m_v):
    wid = lax.axis_index("s") * 2 + lax.axis_index("c")
    pltpu.sync_copy(w_hbm.at[wid], w_v)
    pltpu.sync_copy(wb_hbm.at[wid], wi_v)
    pltpu.sync_copy(j_hbm.at[wid], j_v)
    jv = j_v[...]
    onev = jnp.full((16,), 1, jnp.int32)

    def bit_step(i, ansv):
        candv = ansv | jnp.left_shift(onev, 30 - i)

        def chunk(c, cntv):
            for u in range(_UNROLL):
                wb = wi_v[pl.ds((c * _UNROLL + u) * 16, 16)]
                cntv = cntv + plsc.all_reduce_population_count(wb >= candv)
            return cntv

        cntv = lax.fori_loop(0, _CHUNKS // _UNROLL, chunk,
                             jnp.zeros((16,), jnp.int32))
        return jnp.where(cntv >= jv, candv, ansv)

    ansv = lax.fori_loop(0, 31, bit_step, jnp.zeros((16,), jnp.int32))

    def chunk2(c, carry):
        sacc, caccv = carry
        for u in range(_UNROLL):
            off = (c * _UNROLL + u) * 16
            wb = wi_v[pl.ds(off, 16)]
            gt = wb > ansv
            sacc = sacc + jnp.where(gt, w_v[pl.ds(off, 16)], 0.0)
            caccv = caccv + plsc.all_reduce_population_count(gt)
        return sacc, caccv

    sacc, caccv = lax.fori_loop(
        0, _CHUNKS // _UNROLL, chunk2,
        (jnp.zeros((16,), jnp.float32), jnp.zeros((16,), jnp.int32)))
    o_v[...] = sacc
    m_v[pl.ds(0, 16)] = ansv
    m_v[pl.ds(16, 16)] = caccv
    pltpu.sync_copy(o_v, out_hbm.at[wid])
    pltpu.sync_copy(m_v, meta_hbm.at[wid])


def _topk_pass(w_pad, wb_pad, j2):
    fn = pl.kernel(
        _topk_body,
        out_type=(
            jax.ShapeDtypeStruct((_B, 16), jnp.float32),
            jax.ShapeDtypeStruct((_B, 32), jnp.int32),
        ),
        mesh=plsc.VectorSubcoreMesh(core_axis_name="c", subcore_axis_name="s"),
        compiler_params=pltpu.CompilerParams(needs_layout_passes=False),
        scratch_types=[
            pltpu.VMEM((_P_PAD,), jnp.float32),
            pltpu.VMEM((_P_PAD,), jnp.int32),
            pltpu.VMEM((16,), jnp.int32),
            pltpu.VMEM((16,), jnp.float32),
            pltpu.VMEM((32,), jnp.int32),
        ],
    )
    return fn(w_pad, wb_pad, j2)


@jax.jit
def kernel(loc_data, conf_data, loc_t, conf_t):
    lab = conf_t.astype(jnp.int32)
    lab3 = lab.reshape(_B, 1, _P)
    d4 = jnp.where((lab != 0)[:, :, None], loc_data - loc_t, 0.0)
    d4 = d4.reshape(_B, 1, 4 * _P)
    w3, wb3, stats = _dense_pass(conf_data, lab3, d4)
    stats = stats[:, 0, :]
    w_pad = w3.reshape(_B, _P_PAD)
    wb_pad = wb3.reshape(_B, _P_PAD)
    j = jnp.round(stats[:, 3]).astype(jnp.int32)
    j2 = jnp.broadcast_to(j[:, None], (_B, 16)) + jnp.zeros((_B, 16), jnp.int32)
    srows, meta = _topk_pass(w_pad, wb_pad, j2)
    ans = meta[:, 0]
    cnt = meta[:, 16]
    tau = lax.bitcast_convert_type(ans, jnp.float32)
    s_row = jnp.sum(srows, axis=1) + jnp.where(
        j > cnt, (j - cnt).astype(jnp.float32) * tau, 0.0)
    num_pos = stats[:, 0]
    n = jnp.maximum(jnp.sum(num_pos), 1.0)
    loss_l = jnp.sum(stats[:, 2]) / n
    loss_c = (jnp.sum(stats[:, 1]) + jnp.sum(s_row)) / n
    return (loss_l, loss_c)


# speedup vs baseline: 1.1047x; 1.0266x over previous
import functools

import jax
import jax.numpy as jnp
from jax import lax
from jax.experimental import pallas as pl
from jax.experimental.pallas import tpu as pltpu
from jax.experimental.pallas import tpu_sc as plsc

_B, _P, _C = 32, 8732, 81
_PBLK = 9216
_NP = 1
_BB = 2
_P_PAD = _PBLK * _NP
_CHUNKS = _P_PAD // 16
_UNROLL = 8
_NEGPOS = 3


def _dense_body(conf_ref, lab_ref, d4_ref, w_ref, wb_ref, stats_ref):
    p = pl.program_id(1)
    cols = lax.broadcasted_iota(jnp.int32, (1, _PBLK), 1) + p * _PBLK
    valid = cols < _P
    cols4 = lax.broadcasted_iota(jnp.int32, (1, 4 * _PBLK), 1) + p * 4 * _PBLK
    v4 = cols4 < 4 * _P
    cls = lax.broadcasted_iota(jnp.int32, (_C, _PBLK), 0)
    li1 = lax.broadcasted_iota(jnp.int32, (1, 1, 128), 2)
    parts = []
    for r in range(_BB):
        conf = jnp.transpose(conf_ref[r], (1, 0))
        lab = lab_ref[r]
        m = jnp.max(conf, axis=0, keepdims=True)
        e = jnp.exp(conf - m)
        lse = jnp.log(jnp.sum(e, axis=0, keepdims=True)) + m
        picked = jnp.sum(jnp.where(cls == lab, conf, 0.0), axis=0,
                         keepdims=True)
        ce = lse - picked
        isp = lab != 0
        posm = isp & valid
        w = jnp.where(valid & jnp.logical_not(isp), ce, 0.0)
        w_ref[r] = w
        wb_ref[r] = lax.bitcast_convert_type(w, jnp.int32)

        np_p = jnp.sum(jnp.where(posm, 1.0, 0.0))
        pce_p = jnp.sum(jnp.where(posm, ce, 0.0))
        d = jnp.where(v4, d4_ref[r], 0.0)
        ad = jnp.abs(d)
        sl1 = jnp.where(ad < 1.0, 0.5 * d * d, ad - 0.5)
        ll_p = jnp.sum(sl1)
        parts.append(jnp.where(
            li1 == 0, np_p,
            jnp.where(li1 == 1, pce_p, jnp.where(li1 == 2, ll_p, 0.0))))
    partial = jnp.concatenate(parts, axis=0)

    @pl.when(p == 0)
    def _init():
        stats_ref[...] = jnp.zeros((_BB, 1, 128), jnp.float32)

    stats_ref[...] += partial

    @pl.when(p == _NP - 1)
    def _finish():
        rows = lax.broadcasted_iota(jnp.int32, (_BB, 1, 128), 0)
        li = lax.broadcasted_iota(jnp.int32, (_BB, 1, 128), 2)
        upd = stats_ref[...]
        for r in range(_BB):
            np_i = stats_ref[r, 0, 0].astype(jnp.int32)
            k = jnp.clip(_NEGPOS * np_i, 1, _P - 1)
            j = jnp.minimum(k, _P - np_i)
            upd = jnp.where((rows == r) & (li == 3), j.astype(jnp.float32),
                            upd)
        stats_ref[...] = upd


def _dense_pass(conf_data, lab3, d4):
    return pl.pallas_call(
        _dense_body,
        grid=(_B // _BB, _NP),
        in_specs=[
            pl.BlockSpec((_BB, _PBLK, _C), lambda b, p: (b, p, 0)),
            pl.BlockSpec((_BB, 1, _PBLK), lambda b, p: (b, 0, p)),
            pl.BlockSpec((_BB, 1, 4 * _PBLK), lambda b, p: (b, 0, p)),
        ],
        out_specs=[
            pl.BlockSpec((_BB, 1, _PBLK), lambda b, p: (b, 0, p)),
            pl.BlockSpec((_BB, 1, _PBLK), lambda b, p: (b, 0, p)),
            pl.BlockSpec((_BB, 1, 128), lambda b, p: (b, 0, 0)),
        ],
        out_shape=[
            jax.ShapeDtypeStruct((_B, 1, _P_PAD), jnp.float32),
            jax.ShapeDtypeStruct((_B, 1, _P_PAD), jnp.int32),
            jax.ShapeDtypeStruct((_B, 1, 128), jnp.float32),
        ],
    )(conf_data, lab3, d4)


def _topk_body(w_hbm, wb_hbm, j_hbm, out_hbm, meta_hbm, w_v, wi_v, j_v, o_v, m_v):
    wid = lax.axis_index("s") * 2 + lax.axis_index("c")
    pltpu.sync_copy(w_hbm.at[wid], w_v)
    pltpu.sync_copy(wb_hbm.at[wid], wi_v)
    pltpu.sync_copy(j_hbm.at[wid], j_v)
    jv = j_v[...]
    onev = jnp.full((16,), 1, jnp.int32)

    def bit_step(i, ansv):
        candv = ansv | jnp.left_shift(onev, 30 - i)

        def chunk(c, cntv):
            for u in range(_UNROLL):
                wb = wi_v[pl.ds((c * _UNROLL + u) * 16, 16)]
                cntv = cntv + plsc.all_reduce_population_count(wb >= candv)
            return cntv

        cntv = lax.fori_loop(0, _CHUNKS // _UNROLL, chunk,
                             jnp.zeros((16,), jnp.int32))
        return jnp.where(cntv >= jv, candv, ansv)

    ansv = lax.fori_loop(0, 31, bit_step, jnp.zeros((16,), jnp.int32))

    def chunk2(c, carry):
        sacc, caccv = carry
        for u in range(_UNROLL):
            off = (c * _UNROLL + u) * 16
            wb = wi_v[pl.ds(off, 16)]
            gt = wb > ansv
            sacc = sacc + jnp.where(gt, w_v[pl.ds(off, 16)], 0.0)
            caccv = caccv + plsc.all_reduce_population_count(gt)
        return sacc, caccv

    sacc, caccv = lax.fori_loop(
        0, _CHUNKS // _UNROLL, chunk2,
        (jnp.zeros((16,), jnp.float32), jnp.zeros((16,), jnp.int32)))
    o_v[...] = sacc
    m_v[pl.ds(0, 16)] = ansv
    m_v[pl.ds(16, 16)] = caccv
    pltpu.sync_copy(o_v, out_hbm.at[wid])
    pltpu.sync_copy(m_v, meta_hbm.at[wid])


def _topk_pass(w_pad, wb_pad, j2):
    fn = pl.kernel(
        _topk_body,
        out_type=(
            jax.ShapeDtypeStruct((_B, 16), jnp.float32),
            jax.ShapeDtypeStruct((_B, 32), jnp.int32),
        ),
        mesh=plsc.VectorSubcoreMesh(core_axis_name="c", subcore_axis_name="s"),
        compiler_params=pltpu.CompilerParams(needs_layout_passes=False),
        scratch_types=[
            pltpu.VMEM((_P_PAD,), jnp.float32),
            pltpu.VMEM((_P_PAD,), jnp.int32),
            pltpu.VMEM((16,), jnp.int32),
            pltpu.VMEM((16,), jnp.float32),
            pltpu.VMEM((32,), jnp.int32),
        ],
    )
    return fn(w_pad, wb_pad, j2)


@jax.jit
def kernel(loc_data, conf_data, loc_t, conf_t):
    lab = conf_t.astype(jnp.int32)
    lab3 = lab.reshape(_B, 1, _P)
    d4 = jnp.where((lab != 0)[:, :, None], loc_data - loc_t, 0.0)
    d4 = d4.reshape(_B, 1, 4 * _P)
    w3, wb3, stats = _dense_pass(conf_data, lab3, d4)
    stats = stats[:, 0, :]
    w_pad = w3.reshape(_B, _P_PAD)
    wb_pad = wb3.reshape(_B, _P_PAD)
    j = jnp.round(stats[:, 3]).astype(jnp.int32)
    j2 = jnp.broadcast_to(j[:, None], (_B, 16)) + jnp.zeros((_B, 16), jnp.int32)
    srows, meta = _topk_pass(w_pad, wb_pad, j2)
    ans = meta[:, 0]
    cnt = meta[:, 16]
    tau = lax.bitcast_convert_type(ans, jnp.float32)
    s_row = jnp.sum(srows, axis=1) + jnp.where(
        j > cnt, (j - cnt).astype(jnp.float32) * tau, 0.0)
    num_pos = stats[:, 0]
    n = jnp.maximum(jnp.sum(num_pos), 1.0)
    loss_l = jnp.sum(stats[:, 2]) / n
    loss_c = (jnp.sum(stats[:, 1]) + jnp.sum(s_row)) / n
    return (loss_l, loss_c)


# speedup vs baseline: 1.1071x; 1.0022x over previous
import functools

import jax
import jax.numpy as jnp
from jax import lax
from jax.experimental import pallas as pl
from jax.experimental.pallas import tpu as pltpu
from jax.experimental.pallas import tpu_sc as plsc

_B, _P, _C = 32, 8732, 81
_PBLK = 9216
_NP = 1
_BB = 4
_P_PAD = _PBLK * _NP
_CHUNKS = _P_PAD // 16
_UNROLL = 8
_NEGPOS = 3


def _dense_body(conf_ref, lab_ref, d4_ref, w_ref, wb_ref, stats_ref):
    p = pl.program_id(1)
    cols = lax.broadcasted_iota(jnp.int32, (1, _PBLK), 1) + p * _PBLK
    valid = cols < _P
    cols4 = lax.broadcasted_iota(jnp.int32, (1, 4 * _PBLK), 1) + p * 4 * _PBLK
    v4 = cols4 < 4 * _P
    cls = lax.broadcasted_iota(jnp.int32, (_C, _PBLK), 0)
    li1 = lax.broadcasted_iota(jnp.int32, (1, 1, 128), 2)
    parts = []
    for r in range(_BB):
        conf = jnp.transpose(conf_ref[r], (1, 0))
        lab = lab_ref[r]
        m = jnp.max(conf, axis=0, keepdims=True)
        e = jnp.exp(conf - m)
        lse = jnp.log(jnp.sum(e, axis=0, keepdims=True)) + m
        picked = jnp.sum(jnp.where(cls == lab, conf, 0.0), axis=0,
                         keepdims=True)
        ce = lse - picked
        isp = lab != 0
        posm = isp & valid
        w = jnp.where(valid & jnp.logical_not(isp), ce, 0.0)
        w_ref[r] = w
        wb_ref[r] = lax.bitcast_convert_type(w, jnp.int32)

        np_p = jnp.sum(jnp.where(posm, 1.0, 0.0))
        pce_p = jnp.sum(jnp.where(posm, ce, 0.0))
        d = jnp.where(v4, d4_ref[r], 0.0)
        ad = jnp.abs(d)
        sl1 = jnp.where(ad < 1.0, 0.5 * d * d, ad - 0.5)
        ll_p = jnp.sum(sl1)
        parts.append(jnp.where(
            li1 == 0, np_p,
            jnp.where(li1 == 1, pce_p, jnp.where(li1 == 2, ll_p, 0.0))))
    partial = jnp.concatenate(parts, axis=0)

    @pl.when(p == 0)
    def _init():
        stats_ref[...] = jnp.zeros((_BB, 1, 128), jnp.float32)

    stats_ref[...] += partial

    @pl.when(p == _NP - 1)
    def _finish():
        rows = lax.broadcasted_iota(jnp.int32, (_BB, 1, 128), 0)
        li = lax.broadcasted_iota(jnp.int32, (_BB, 1, 128), 2)
        upd = stats_ref[...]
        for r in range(_BB):
            np_i = stats_ref[r, 0, 0].astype(jnp.int32)
            k = jnp.clip(_NEGPOS * np_i, 1, _P - 1)
            j = jnp.minimum(k, _P - np_i)
            upd = jnp.where((rows == r) & (li == 3), j.astype(jnp.float32),
                            upd)
        stats_ref[...] = upd


def _dense_pass(conf_data, lab3, d4):
    return pl.pallas_call(
        _dense_body,
        grid=(_B // _BB, _NP),
        in_specs=[
            pl.BlockSpec((_BB, _PBLK, _C), lambda b, p: (b, p, 0)),
            pl.BlockSpec((_BB, 1, _PBLK), lambda b, p: (b, 0, p)),
            pl.BlockSpec((_BB, 1, 4 * _PBLK), lambda b, p: (b, 0, p)),
        ],
        out_specs=[
            pl.BlockSpec((_BB, 1, _PBLK), lambda b, p: (b, 0, p)),
            pl.BlockSpec((_BB, 1, _PBLK), lambda b, p: (b, 0, p)),
            pl.BlockSpec((_BB, 1, 128), lambda b, p: (b, 0, 0)),
        ],
        out_shape=[
            jax.ShapeDtypeStruct((_B, 1, _P_PAD), jnp.float32),
            jax.ShapeDtypeStruct((_B, 1, _P_PAD), jnp.int32),
            jax.ShapeDtypeStruct((_B, 1, 128), jnp.float32),
        ],
    )(conf_data, lab3, d4)


def _topk_body(w_hbm, wb_hbm, j_hbm, out_hbm, meta_hbm, w_v, wi_v, j_v, o_v, m_v):
    wid = lax.axis_index("s") * 2 + lax.axis_index("c")
    pltpu.sync_copy(w_hbm.at[wid], w_v)
    pltpu.sync_copy(wb_hbm.at[wid], wi_v)
    pltpu.sync_copy(j_hbm.at[wid], j_v)
    jv = j_v[...]
    onev = jnp.full((16,), 1, jnp.int32)

    def bit_step(i, ansv):
        candv = ansv | jnp.left_shift(onev, 30 - i)

        def chunk(c, cntv):
            for u in range(_UNROLL):
                wb = wi_v[pl.ds((c * _UNROLL + u) * 16, 16)]
                cntv = cntv + plsc.all_reduce_population_count(wb >= candv)
            return cntv

        cntv = lax.fori_loop(0, _CHUNKS // _UNROLL, chunk,
                             jnp.zeros((16,), jnp.int32))
        return jnp.where(cntv >= jv, candv, ansv)

    ansv = lax.fori_loop(0, 31, bit_step, jnp.zeros((16,), jnp.int32))

    def chunk2(c, carry):
        sacc, caccv = carry
        for u in range(_UNROLL):
            off = (c * _UNROLL + u) * 16
            wb = wi_v[pl.ds(off, 16)]
            gt = wb > ansv
            sacc = sacc + jnp.where(gt, w_v[pl.ds(off, 16)], 0.0)
            caccv = caccv + plsc.all_reduce_population_count(gt)
        return sacc, caccv

    sacc, caccv = lax.fori_loop(
        0, _CHUNKS // _UNROLL, chunk2,
        (jnp.zeros((16,), jnp.float32), jnp.zeros((16,), jnp.int32)))
    o_v[...] = sacc
    m_v[pl.ds(0, 16)] = ansv
    m_v[pl.ds(16, 16)] = caccv
    pltpu.sync_copy(o_v, out_hbm.at[wid])
    pltpu.sync_copy(m_v, meta_hbm.at[wid])


def _topk_pass(w_pad, wb_pad, j2):
    fn = pl.kernel(
        _topk_body,
        out_type=(
            jax.ShapeDtypeStruct((_B, 16), jnp.float32),
            jax.ShapeDtypeStruct((_B, 32), jnp.int32),
        ),
        mesh=plsc.VectorSubcoreMesh(core_axis_name="c", subcore_axis_name="s"),
        compiler_params=pltpu.CompilerParams(needs_layout_passes=False),
        scratch_types=[
            pltpu.VMEM((_P_PAD,), jnp.float32),
            pltpu.VMEM((_P_PAD,), jnp.int32),
            pltpu.VMEM((16,), jnp.int32),
            pltpu.VMEM((16,), jnp.float32),
            pltpu.VMEM((32,), jnp.int32),
        ],
    )
    return fn(w_pad, wb_pad, j2)


@jax.jit
def kernel(loc_data, conf_data, loc_t, conf_t):
    lab = conf_t.astype(jnp.int32)
    lab3 = lab.reshape(_B, 1, _P)
    d4 = jnp.where((lab != 0)[:, :, None], loc_data - loc_t, 0.0)
    d4 = d4.reshape(_B, 1, 4 * _P)
    w3, wb3, stats = _dense_pass(conf_data, lab3, d4)
    stats = stats[:, 0, :]
    w_pad = w3.reshape(_B, _P_PAD)
    wb_pad = wb3.reshape(_B, _P_PAD)
    j = jnp.round(stats[:, 3]).astype(jnp.int32)
    j2 = jnp.broadcast_to(j[:, None], (_B, 16)) + jnp.zeros((_B, 16), jnp.int32)
    srows, meta = _topk_pass(w_pad, wb_pad, j2)
    ans = meta[:, 0]
    cnt = meta[:, 16]
    tau = lax.bitcast_convert_type(ans, jnp.float32)
    s_row = jnp.sum(srows, axis=1) + jnp.where(
        j > cnt, (j - cnt).astype(jnp.float32) * tau, 0.0)
    num_pos = stats[:, 0]
    n = jnp.maximum(jnp.sum(num_pos), 1.0)
    loss_l = jnp.sum(stats[:, 2]) / n
    loss_c = (jnp.sum(stats[:, 1]) + jnp.sum(s_row)) / n
    return (loss_l, loss_c)
